# Initial kernel scaffold; baseline (speedup 1.0000x reference)
import functools

import jax
import jax.numpy as jnp
from jax import lax
from jax.experimental import pallas as pl
from jax.experimental.pallas import tpu as pltpu
from jax.experimental.pallas import tpu_sc as plsc

N = 320000
D = 128
NSEG = 10000
SPAD = 10016

NC = 2
NSUB = 16
NW = NC * NSUB
RPT = N // NW
BLK = 250
NBUF = 2
NBLK = RPT // BLK
STAGE = 128
LANES = 16
ZCH = SPAD // NSUB


def _sc_partial_sums(values, seg_ids):
    mesh = plsc.VectorSubcoreMesh(core_axis_name="c", subcore_axis_name="s")

    @functools.partial(
        pl.kernel,
        out_type=jax.ShapeDtypeStruct((NC, NSEG, D), jnp.float32),
        mesh=mesh,
        scratch_types=[
            pltpu.VMEM((NBUF, BLK, D), jnp.float32),
            pltpu.VMEM((RPT,), jnp.int32),
            pltpu.VMEM((STAGE, D), jnp.float32),
            pltpu.VMEM((STAGE,), jnp.int32),
            pltpu.VMEM_SHARED((SPAD, D), jnp.float32),
            pltpu.SemaphoreType.DMA,
            pltpu.SemaphoreType.DMA,
            pltpu.SemaphoreType.DMA,
        ],
    )
    def sc_kernel(vals_hbm, ids_hbm, out_hbm, vbuf, ids_v, stage_v, stage_i,
                  acc_sh, sem0, sem1, isem):
        cid = lax.axis_index("c")
        sid = lax.axis_index("s")
        wid = cid * NSUB + sid
        row0 = wid * RPT
        vsems = (sem0, sem1)
        lane = lax.iota(jnp.int32, LANES)
        zvec = jnp.zeros((LANES,), jnp.float32)

        @pl.loop(0, STAGE)
        def _(r):
            ridx = jnp.full((LANES,), r, jnp.int32)
            for j in range(D // LANES):
                plsc.store_scatter(stage_v, [ridx, j * LANES + lane], zvec)

        z0 = sid * ZCH
        zoff = 0
        while zoff < ZCH:
            cnt = min(STAGE, ZCH - zoff)
            pltpu.sync_copy(stage_v.at[pl.ds(0, cnt)],
                            acc_sh.at[pl.ds(z0 + zoff, cnt)])
            zoff += cnt
        plsc.subcore_barrier()

        pltpu.async_copy(ids_hbm.at[pl.ds(row0, RPT)], ids_v, isem).wait()
        for b in range(NBUF):
            pltpu.async_copy(vals_hbm.at[pl.ds(row0 + b * BLK, BLK)],
                             vbuf.at[b], vsems[b])

        def emit(k, seg, accs):
            ridx = jnp.full((LANES,), k, jnp.int32)
            for j in range(D // LANES):
                plsc.store_scatter(stage_v, [ridx, j * LANES + lane], accs[j])
            plsc.store_scatter(stage_i, [ridx],
                               jnp.full((LANES,), seg, jnp.int32),
                               mask=lane == 0)
            kn = k + 1

            def flush():
                pltpu.sync_copy(stage_v, acc_sh.at[stage_i], add=True)
                return jnp.int32(0)

            return lax.cond(kn == STAGE, flush, lambda: kn)

        def row_step(r, c, vb, base):
            k, prev = c[0], c[1]
            accs = c[2:]
            seg = ids_v[base + r]
            boundary = seg != prev
            k = lax.cond(boundary, lambda: emit(k, prev, accs), lambda: k)
            new_accs = []
            for j in range(D // LANES):
                rowj = vb[r, pl.ds(j * LANES, LANES)]
                new_accs.append(jnp.where(boundary, rowj, accs[j] + rowj))
            return (k, seg) + tuple(new_accs)

        def outer(g, c):
            for b in range(NBUF):
                blk = g * NBUF + b
                pltpu.make_async_copy(vals_hbm.at[pl.ds(0, BLK)], vbuf.at[b],
                                      vsems[b]).wait()
                base = blk * BLK
                c = lax.fori_loop(
                    0, BLK, lambda r, cc: row_step(r, cc, vbuf.at[b], base), c)
                nxt = blk + NBUF

                @pl.when(nxt < NBLK)
                def _():
                    pltpu.async_copy(
                        vals_hbm.at[pl.ds(row0 + nxt * BLK, BLK)],
                        vbuf.at[b], vsems[b])
            return c

        carry0 = (jnp.int32(0), ids_v[0]) + (zvec,) * (D // LANES)
        carry = lax.fori_loop(0, NBLK // NBUF, outer, carry0)

        k = emit(carry[0], carry[1], carry[2:])
        dummy = jnp.full((LANES,), NSEG, jnp.int32)
        for j in range(STAGE // LANES):
            cur = stage_i[pl.ds(j * LANES, LANES)]
            stage_i[pl.ds(j * LANES, LANES)] = jnp.where(
                j * LANES + lane >= k, dummy, cur)
        pltpu.sync_copy(stage_v, acc_sh.at[stage_i], add=True)

        plsc.subcore_barrier()
        orows = NSEG // NSUB
        pltpu.sync_copy(acc_sh.at[pl.ds(sid * orows, orows)],
                        out_hbm.at[cid, pl.ds(sid * orows, orows)])

    return sc_kernel(values, seg_ids)


def _combine_body(p_ref, o_ref):
    o_ref[...] = p_ref[0] + p_ref[1]


def _tc_combine(partials):
    return pl.pallas_call(
        _combine_body,
        out_shape=jax.ShapeDtypeStruct((NSEG, D), jnp.float32),
    )(partials)


def kernel(values, segment_ids):
    ids = segment_ids.astype(jnp.int32)
    partials = _sc_partial_sums(values, ids)
    return _tc_combine(partials)


# speedup vs baseline: 3.4272x; 3.4272x over previous
import dataclasses
import functools

import jax
import jax.numpy as jnp
from jax import lax
from jax.experimental import pallas as pl
from jax.experimental.pallas import tpu as pltpu
from jax.experimental.pallas import tpu_sc as plsc

N = 320000
D = 128
NSEG = 10000
SPAD = 10016

NC = 2
NSUB = 16
NW = NC * NSUB
RPT = N // NW
BLK = 80
NBUF = 2
NBLK = RPT // BLK
STAGE = 128
LANES = 16
NJ = D // LANES


def _sc_partial_sums(values, seg_ids):
    mesh = plsc.VectorSubcoreMesh(core_axis_name="c", subcore_axis_name="s")
    cp = pltpu.CompilerParams()
    if "needs_layout_passes" in pltpu.CompilerParams.__dataclass_fields__:
        cp = dataclasses.replace(cp, needs_layout_passes=False)

    @functools.partial(
        pl.kernel,
        compiler_params=cp,
        out_type=jax.ShapeDtypeStruct((NC, NSEG, D), jnp.float32),
        mesh=mesh,
        scratch_types=[
            pltpu.VMEM((NBUF, BLK, D), jnp.float32),
            pltpu.VMEM((NBUF, BLK + LANES), jnp.int32),
            pltpu.VMEM((STAGE, D), jnp.float32),
            pltpu.VMEM((STAGE,), jnp.int32),
            pltpu.VMEM_SHARED((SPAD, D), jnp.float32),
            pltpu.SemaphoreType.DMA,
            pltpu.SemaphoreType.DMA,
        ],
    )
    def sc_kernel(vals_hbm, ids_hbm, out_hbm, vbuf, ibuf, stage_v, stage_i,
                  acc_sh, sem0, sem1):
        cid = lax.axis_index("c")
        sid = lax.axis_index("s")
        wid = cid * NSUB + sid
        row0 = wid * RPT
        sems = (sem0, sem1)
        lane = lax.iota(jnp.int32, LANES)
        zvec = jnp.zeros((LANES,), jnp.float32)

        @pl.loop(0, STAGE)
        def _(r):
            ridx = jnp.full((LANES,), r, jnp.int32)
            for j in range(NJ):
                plsc.store_scatter(stage_v, [ridx, j * LANES + lane], zvec)

        zch = NSEG // NSUB
        z0 = sid * zch
        zoff = 0
        while zoff < zch:
            cnt = min(STAGE, zch - zoff)
            pltpu.sync_copy(stage_v.at[pl.ds(0, cnt)],
                            acc_sh.at[pl.ds(z0 + zoff, cnt)])
            zoff += cnt
        plsc.subcore_barrier()

        def start_fetch(blk, b):
            pltpu.async_copy(vals_hbm.at[pl.ds(row0 + blk * BLK, BLK)],
                             vbuf.at[b], sems[b])
            pltpu.async_copy(ids_hbm.at[pl.ds(row0 + blk * BLK, BLK)],
                             ibuf.at[b, pl.ds(0, BLK)], sems[b])

        for b in range(NBUF):
            start_fetch(b, b)

        def emit(k, seg, accs):
            ridx = jnp.full((LANES,), k, jnp.int32)
            for j in range(NJ):
                plsc.store_scatter(stage_v, [ridx, j * LANES + lane], accs[j])
            seg = jnp.where(seg < 0, NSEG, seg)
            plsc.store_scatter(stage_i, [ridx],
                               jnp.full((LANES,), seg, jnp.int32),
                               mask=lane == 0)
            kn = k + 1

            def flush():
                pltpu.sync_copy(stage_v, acc_sh.at[stage_i], add=True)
                return jnp.int32(0)

            return lax.cond(kn == STAGE, flush, lambda: kn)

        def row_step(r, c, vb, ib):
            k, prev = c[0], c[1]
            accs = c[2:]
            seg = ib[pl.ds(r, LANES)][0]
            boundary = seg != prev
            k = lax.cond(boundary, lambda: emit(k, prev, accs), lambda: k)
            new_accs = []
            for j in range(NJ):
                rowj = vb[r, pl.ds(j * LANES, LANES)]
                new_accs.append(jnp.where(boundary, rowj, accs[j] + rowj))
            return (k, seg) + tuple(new_accs)

        def process_block(blk, b, c):
            pltpu.make_async_copy(vals_hbm.at[pl.ds(0, BLK)], vbuf.at[b],
                                  sems[b]).wait()
            pltpu.make_async_copy(ids_hbm.at[pl.ds(0, BLK)],
                                  ibuf.at[b, pl.ds(0, BLK)], sems[b]).wait()
            c = lax.fori_loop(
                0, BLK,
                lambda r, cc: row_step(r, cc, vbuf.at[b], ibuf.at[b]), c)
            nxt = blk + NBUF

            @pl.when(nxt < NBLK)
            def _():
                start_fetch(nxt, b)
            return c

        def outer(g, c):
            for b in range(NBUF):
                c = process_block(g * NBUF + b, b, c)
            return c

        carry0 = (jnp.int32(0), jnp.int32(-1)) + (zvec,) * NJ
        carry = lax.fori_loop(0, NBLK // NBUF, outer, carry0)
        if NBLK % NBUF:
            carry = process_block(jnp.int32(NBLK - 1), 0, carry)

        k = emit(carry[0], carry[1], carry[2:])
        dummy = jnp.full((LANES,), NSEG, jnp.int32)
        for j in range(STAGE // LANES):
            cur = stage_i[pl.ds(j * LANES, LANES)]
            stage_i[pl.ds(j * LANES, LANES)] = jnp.where(
                j * LANES + lane >= k, dummy, cur)
        pltpu.sync_copy(stage_v, acc_sh.at[stage_i], add=True)

        plsc.subcore_barrier()
        @pl.when(sid < NSUB - 1)
        def _():
            pltpu.sync_copy(acc_sh.at[pl.ds(sid * 624, 624)],
                            out_hbm.at[cid, pl.ds(sid * 624, 624)])

        @pl.when(sid == NSUB - 1)
        def _():
            tail = NSEG - 624 * (NSUB - 1)
            pltpu.sync_copy(acc_sh.at[pl.ds(624 * (NSUB - 1), tail)],
                            out_hbm.at[cid, pl.ds(624 * (NSUB - 1), tail)])

    return sc_kernel(values, seg_ids)


def _combine_body(p_ref, o_ref):
    o_ref[...] = p_ref[0] + p_ref[1]


def _tc_combine(partials):
    return pl.pallas_call(
        _combine_body,
        out_shape=jax.ShapeDtypeStruct((NSEG, D), jnp.float32),
    )(partials)


def kernel(values, segment_ids):
    ids = segment_ids.astype(jnp.int32)
    partials = _sc_partial_sums(values, ids)
    return _tc_combine(partials)


# speedup vs baseline: 4.0512x; 1.1821x over previous
import dataclasses
import functools

import jax
import jax.numpy as jnp
from jax import lax
from jax.experimental import pallas as pl
from jax.experimental.pallas import tpu as pltpu
from jax.experimental.pallas import tpu_sc as plsc

N = 320000
D = 128
NSEG = 10000
SPAD = 10016

NC = 2
NSUB = 16
NW = NC * NSUB
RPT = N // NW
BLK = 80
NBUF = 2
NBLK = RPT // BLK
STAGE = 128
LANES = 16
NJ = D // LANES


def _sc_partial_sums(values, seg_ids):
    mesh = plsc.VectorSubcoreMesh(core_axis_name="c", subcore_axis_name="s")
    cp = pltpu.CompilerParams()
    if "needs_layout_passes" in pltpu.CompilerParams.__dataclass_fields__:
        cp = dataclasses.replace(cp, needs_layout_passes=False)

    @functools.partial(
        pl.kernel,
        compiler_params=cp,
        out_type=jax.ShapeDtypeStruct((NC, NSEG, D), jnp.float32),
        mesh=mesh,
        scratch_types=[
            pltpu.VMEM((NBUF, BLK, D), jnp.float32),
            pltpu.VMEM((NBUF, BLK), jnp.int32),
            pltpu.VMEM((STAGE, D), jnp.float32),
            pltpu.VMEM((STAGE,), jnp.int32),
            pltpu.VMEM_SHARED((SPAD, D), jnp.float32),
            pltpu.SemaphoreType.DMA,
            pltpu.SemaphoreType.DMA,
        ],
    )
    def sc_kernel(vals_hbm, ids_hbm, out_hbm, vbuf, ibuf, stage_v, stage_i,
                  acc_sh, sem0, sem1):
        cid = lax.axis_index("c")
        sid = lax.axis_index("s")
        wid = cid * NSUB + sid
        row0 = wid * RPT
        sems = (sem0, sem1)
        lane = lax.iota(jnp.int32, LANES)
        zvec = jnp.zeros((LANES,), jnp.float32)

        @pl.loop(0, STAGE)
        def _(r):
            ridx = jnp.full((LANES,), r, jnp.int32)
            for j in range(NJ):
                plsc.store_scatter(stage_v, [ridx, j * LANES + lane], zvec)

        zch = NSEG // NSUB
        z0 = sid * zch
        zoff = 0
        while zoff < zch:
            cnt = min(STAGE, zch - zoff)
            pltpu.sync_copy(stage_v.at[pl.ds(0, cnt)],
                            acc_sh.at[pl.ds(z0 + zoff, cnt)])
            zoff += cnt
        plsc.subcore_barrier()

        def start_fetch(blk, b):
            pltpu.async_copy(vals_hbm.at[pl.ds(row0 + blk * BLK, BLK)],
                             vbuf.at[b], sems[b])
            pltpu.async_copy(ids_hbm.at[pl.ds(row0 + blk * BLK, BLK)],
                             ibuf.at[b, pl.ds(0, BLK)], sems[b])

        for b in range(NBUF):
            start_fetch(b, b)

        def emit(k, seg, accs, snaps):
            ridx = jnp.full((LANES,), k, jnp.int32)
            for j in range(NJ):
                plsc.store_scatter(stage_v, [ridx, j * LANES + lane],
                                   accs[j] - snaps[j])
            seg = jnp.where(seg < 0, NSEG, seg)
            plsc.store_scatter(stage_i, [ridx],
                               jnp.full((LANES,), seg, jnp.int32),
                               mask=lane == 0)
            kn = k + 1

            def flush():
                pltpu.sync_copy(stage_v, acc_sh.at[stage_i], add=True)
                return jnp.int32(0)

            return lax.cond(kn == STAGE, flush, lambda: kn)

        def group_step(g, c, vb, ib):
            k, prev = c[0], c[1]
            snaps = c[2:2 + NJ]
            accs = c[2 + NJ:]
            idv = ib[pl.ds(g * LANES, LANES)]
            for i in range(LANES):
                seg = idv[i]
                boundary = seg != prev
                res = lax.cond(
                    boundary,
                    lambda k=k, prev=prev, accs=accs, snaps=snaps:
                        (emit(k, prev, accs, snaps),) + accs,
                    lambda k=k, snaps=snaps: (k,) + snaps)
                k, snaps = res[0], tuple(res[1:])
                prev = seg
                accs = tuple(
                    accs[j] + vb[g * LANES + i, pl.ds(j * LANES, LANES)]
                    for j in range(NJ))
            return (k, prev) + snaps + accs

        def process_block(blk, b, c):
            pltpu.make_async_copy(vals_hbm.at[pl.ds(0, BLK)], vbuf.at[b],
                                  sems[b]).wait()
            pltpu.make_async_copy(ids_hbm.at[pl.ds(0, BLK)],
                                  ibuf.at[b, pl.ds(0, BLK)], sems[b]).wait()
            c = lax.fori_loop(
                0, BLK // LANES,
                lambda g, cc: group_step(g, cc, vbuf.at[b], ibuf.at[b]), c)
            nxt = blk + NBUF

            @pl.when(nxt < NBLK)
            def _():
                start_fetch(nxt, b)
            return c

        def outer(g, c):
            for b in range(NBUF):
                c = process_block(g * NBUF + b, b, c)
            return c

        carry0 = (jnp.int32(0), jnp.int32(-1)) + (zvec,) * (2 * NJ)
        carry = lax.fori_loop(0, NBLK // NBUF, outer, carry0)
        if NBLK % NBUF:
            carry = process_block(jnp.int32(NBLK - 1), 0, carry)

        k = emit(carry[0], carry[1], carry[2 + NJ:], carry[2:2 + NJ])
        dummy = jnp.full((LANES,), NSEG, jnp.int32)
        for j in range(STAGE // LANES):
            cur = stage_i[pl.ds(j * LANES, LANES)]
            stage_i[pl.ds(j * LANES, LANES)] = jnp.where(
                j * LANES + lane >= k, dummy, cur)
        pltpu.sync_copy(stage_v, acc_sh.at[stage_i], add=True)

        plsc.subcore_barrier()
        @pl.when(sid < NSUB - 1)
        def _():
            pltpu.sync_copy(acc_sh.at[pl.ds(sid * 624, 624)],
                            out_hbm.at[cid, pl.ds(sid * 624, 624)])

        @pl.when(sid == NSUB - 1)
        def _():
            tail = NSEG - 624 * (NSUB - 1)
            pltpu.sync_copy(acc_sh.at[pl.ds(624 * (NSUB - 1), tail)],
                            out_hbm.at[cid, pl.ds(624 * (NSUB - 1), tail)])

    return sc_kernel(values, seg_ids)


def _combine_body(p_ref, o_ref):
    o_ref[...] = p_ref[0] + p_ref[1]


def _tc_combine(partials):
    return pl.pallas_call(
        _combine_body,
        out_shape=jax.ShapeDtypeStruct((NSEG, D), jnp.float32),
    )(partials)


def kernel(values, segment_ids):
    ids = segment_ids.astype(jnp.int32)
    partials = _sc_partial_sums(values, ids)
    return _tc_combine(partials)
